# baseline (speedup 1.0000x reference)
import jax
import jax.numpy as jnp
from jax import lax
from jax.experimental import pallas as pl
from jax.experimental.pallas import tpu as pltpu

_THRESHOLD = 0.5
_NEG_POS_RATIO = 3.0
_F32_INF_BITS = 0x7F800000
_HI = lax.Precision.HIGHEST


def _iota2(shape, dim):
    return lax.broadcasted_iota(jnp.int32, shape, dim)


def _mbl_kernel(scores_ref, locs_ref, payload_ref, boxes_ref, priors_ref,
                out_ref, conf_ref, npos_ref, cpos_ref, labs_ref):
    b = pl.program_id(0)
    nb = pl.num_programs(0)

    nq, nj = scores_ref.shape[1], scores_ref.shape[2]
    ns = 8
    nc = nj // ns
    num_obj = boxes_ref.shape[1]
    num_pri = nq * ns

    scores_t = jnp.transpose(scores_ref[0])
    locs_t = jnp.transpose(locs_ref[0])
    priors_t = jnp.transpose(priors_ref[...])

    rp = _iota2((32, 32), 0)
    cp = _iota2((32, 32), 1)
    cperm = (cp == (rp % 8) * 4 + rp // 8).astype(jnp.float32)
    pri = jnp.dot(cperm, priors_t, precision=_HI)
    plo = jnp.dot(cperm, locs_t, precision=_HI)
    pc_x, pc_y = pri[0:8], pri[8:16]
    pw, ph = pri[16:24], pri[24:32]

    px0 = pc_x - pw / 2
    py0 = pc_y - ph / 2
    px1 = pc_x + pw / 2
    py1 = pc_y + ph / 2
    pa = (px1 - px0) * (py1 - py0)

    boxes3 = boxes_ref[0]
    bx0 = boxes3[:, :, 0:1]
    by0 = boxes3[:, :, 1:2]
    bx1 = boxes3[:, :, 2:3]
    by1 = boxes3[:, :, 3:4]
    iw = jnp.clip(jnp.minimum(bx1, px1) - jnp.maximum(bx0, px0), 0.0, None)
    ih = jnp.clip(jnp.minimum(by1, py1) - jnp.maximum(by0, py0), 0.0, None)
    inter = iw * ih
    a1 = (bx1 - bx0) * (by1 - by0)
    union = a1 + pa - inter
    iou = inter / union

    vmax = jnp.max(iou, axis=0)
    onehot = (iou == vmax).astype(jnp.float32).reshape(num_obj * ns, nq)

    payload = payload_ref[0]
    r40 = _iota2((40, 8), 0)
    e40 = (_iota2((40, 8), 1) == r40 // 8).astype(jnp.float32)
    v40 = jnp.dot(e40, payload, precision=_HI)
    rpat = (_iota2((num_obj, num_obj * ns), 1) // 8
            == _iota2((num_obj, num_obj * ns), 0)).astype(jnp.float32)
    v256 = jnp.dot(v40, rpat, precision=_HI)
    amask = (_iota2((40, num_obj * ns), 1) % 8
             == _iota2((40, num_obj * ns), 0) % 8)
    a40 = jnp.where(amask, v256, 0.0)
    g = jnp.dot(a40, onehot, precision=_HI)
    gx0, gy0, gx1, gy1, glab = g[0:8], g[8:16], g[16:24], g[24:32], g[32:40]

    pos = vmax >= _THRESHOLD
    label_i = jnp.where(pos, (glab + 0.5).astype(jnp.int32), 0)
    posf = pos.astype(jnp.float32)
    n_pos = jnp.sum(posf, axis=1, keepdims=True)

    bcx = (gx1 + gx0) / 2
    bcy = (gy1 + gy0) / 2
    bw = gx1 - gx0
    bh = gy1 - gy0
    t0 = (bcx - pc_x) / (pw / 10)
    t1 = (bcy - pc_y) / (ph / 10)
    t2 = jnp.log(bw / pw) * 5
    t3 = jnp.log(bh / ph) * 5
    labs = jnp.sum((jnp.abs(plo[0:8] - t0) + jnp.abs(plo[8:16] - t1)
                    + jnp.abs(plo[16:24] - t2) + jnp.abs(plo[24:32] - t3))
                   * posf, axis=1, keepdims=True)

    e = jnp.exp(scores_t)
    msum = (_iota2((ns, nj), 1) // nc
            == _iota2((ns, nj), 0)).astype(jnp.float32)
    s = jnp.dot(msum, e, precision=_HI)
    lse = jnp.log(s)

    m168 = (_iota2((nj, ns), 0) // nc
            == _iota2((nj, ns), 1)).astype(jnp.float32)
    lab168 = jnp.dot(m168, label_i.astype(jnp.float32), precision=_HI)
    ci168 = _iota2((nj, nq), 0) % nc
    sel = jnp.where(ci168 == (lab168 + 0.5).astype(jnp.int32),
                    scores_t, 0.0)
    x_t = jnp.dot(msum, sel, precision=_HI)
    conf_all = lse - x_t

    cpos = jnp.sum(conf_all * posf, axis=1, keepdims=True)
    conf_neg = jnp.maximum(jnp.where(pos, 0.0, conf_all), 0.0)

    conf_ref[pl.ds(ns * b, ns), :] = conf_neg
    npos_ref[pl.ds(ns * b, ns), :] = n_pos
    cpos_ref[pl.ds(ns * b, ns), :] = cpos
    labs_ref[pl.ds(ns * b, ns), :] = labs

    @pl.when(b == nb - 1)
    def _():
        v = conf_ref[...]
        vi = lax.bitcast_convert_type(v, jnp.int32)

        nr = nb * ns
        q256 = (_iota2((nr, nr), 0) // ns
                == _iota2((nr, nr), 1) // ns).astype(jnp.float32)
        ones_col = jnp.zeros((nq, 1), jnp.float32) + 1.0

        def gsum(rowsum):
            return jnp.dot(q256, rowsum, precision=_HI)

        k8 = jnp.minimum(gsum(npos_ref[...]) * _NEG_POS_RATIO,
                         float(num_pri))
        lo0 = jnp.zeros((nr, 1), jnp.int32)
        hi0 = jnp.full((nr, 1), _F32_INF_BITS, jnp.int32)

        def body(_, carry):
            lo, hi = carry
            mid = lo + (hi - lo) // 2
            rs = jnp.dot((vi >= mid).astype(jnp.float32), ones_col)
            ge = gsum(rs) >= k8
            return jnp.where(ge, mid, lo), jnp.where(ge, hi, mid)

        lo, _hi = lax.fori_loop(0, 31, body, (lo0, hi0))
        tau = lax.bitcast_convert_type(lo, jnp.float32)
        gt = vi > lo
        cnt_gt = gsum(jnp.dot(gt.astype(jnp.float32), ones_col))
        sum_gt = gsum(jnp.sum(jnp.where(gt, v, 0.0), axis=1,
                              keepdims=True))
        top_k_sum = sum_gt + (k8 - cnt_gt) * tau
        top_k_sum = jnp.where(k8 > 0, top_k_sum, 0.0)

        npos_tot = jnp.sum(npos_ref[...])
        conf_loss = ((jnp.sum(top_k_sum) / ns + jnp.sum(cpos_ref[...]))
                     / jnp.maximum(npos_tot, 1.0))
        loc_loss = jnp.sum(labs_ref[...]) / jnp.maximum(npos_tot * 4.0, 1.0)
        out_ref[...] = (conf_loss + loc_loss).reshape(1, 1)


@jax.jit
def kernel(predicted_locs, predicted_scores, boxes, labels, priors_cxcy):
    B, P, NC = predicted_scores.shape
    NO = boxes.shape[1]
    NQ = P // 8

    scores_r = predicted_scores.reshape(B, NQ, 8 * NC)
    locs_r = predicted_locs.reshape(B, NQ, 32)
    priors_r = priors_cxcy.reshape(NQ, 32)
    boxes3 = boxes.reshape(B, NO, 1, 4)
    payload = jnp.concatenate(
        [boxes, labels.astype(jnp.float32)[..., None],
         jnp.zeros((B, NO, 3), jnp.float32)], axis=-1)
    payload_t = jnp.transpose(payload, (0, 2, 1))

    out = pl.pallas_call(
        _mbl_kernel,
        grid=(B,),
        in_specs=[
            pl.BlockSpec((1, NQ, 8 * NC), lambda b: (b, 0, 0)),
            pl.BlockSpec((1, NQ, 32), lambda b: (b, 0, 0)),
            pl.BlockSpec((1, 8, NO), lambda b: (b, 0, 0)),
            pl.BlockSpec((1, NO, 1, 4), lambda b: (b, 0, 0, 0)),
            pl.BlockSpec((NQ, 32), lambda b: (0, 0)),
        ],
        out_specs=pl.BlockSpec((1, 1), lambda b: (0, 0)),
        out_shape=jax.ShapeDtypeStruct((1, 1), jnp.float32),
        scratch_shapes=[
            pltpu.VMEM((B * 8, NQ), jnp.float32),
            pltpu.VMEM((B * 8, 1), jnp.float32),
            pltpu.VMEM((B * 8, 1), jnp.float32),
            pltpu.VMEM((B * 8, 1), jnp.float32),
        ],
        compiler_params=pltpu.CompilerParams(
            dimension_semantics=("arbitrary",)),
    )(scores_r, locs_r, payload_t, boxes3, priors_r)
    return out[0, 0]


# speedup vs baseline: 3.2811x; 3.2811x over previous
import jax
import jax.numpy as jnp
from jax import lax
from jax.experimental import pallas as pl
from jax.experimental.pallas import tpu as pltpu

_THRESHOLD = 0.5
_NEG_POS_RATIO = 3.0
_F32_INF_BITS = 0x7F800000


def _mbl_kernel(scores_ref, locs_ref, payload_ref, boxes_ref, priors_ref,
                out_ref, conf_ref, npos_ref, cpos_ref, labs_ref):
    b = pl.program_id(0)
    nb = pl.num_programs(0)

    scores = scores_ref[0]
    plocs = locs_ref[0]
    payload = payload_ref[0]
    boxes = boxes_ref[0]
    num_obj, _ = boxes.shape
    num_cls, num_pri = scores.shape

    pc = priors_ref[0:2, :]
    pwh = priors_ref[2:4, :]
    pcorner0 = pc - pwh / 2
    pcorner1 = pc + pwh / 2
    px0 = pcorner0[0:1, :]
    py0 = pcorner0[1:2, :]
    px1 = pcorner1[0:1, :]
    py1 = pcorner1[1:2, :]
    pa = (px1 - px0) * (py1 - py0)

    bx0 = boxes[:, 0:1]
    by0 = boxes[:, 1:2]
    bx1 = boxes[:, 2:3]
    by1 = boxes[:, 3:4]
    iw = jnp.clip(jnp.minimum(bx1, px1) - jnp.maximum(bx0, px0), 0.0, None)
    ih = jnp.clip(jnp.minimum(by1, py1) - jnp.maximum(by0, py0), 0.0, None)
    inter = iw * ih
    a1 = (bx1 - bx0) * (by1 - by0)
    union = a1 + pa - inter
    iou = inter / union

    vmax = jnp.max(iou, axis=0, keepdims=True)
    onehot = (iou == vmax).astype(jnp.float32)

    g = jnp.dot(payload, onehot, precision=lax.Precision.HIGHEST)
    pos = vmax >= _THRESHOLD
    label_i = jnp.where(pos, (g[4:5] + 0.5).astype(jnp.int32), 0)
    posf = pos.astype(jnp.float32)
    n_pos = jnp.sum(posf)

    gc0 = g[0:2]
    gc1 = g[2:4]
    bcxy = (gc1 + gc0) / 2
    bwh = gc1 - gc0
    t01 = (bcxy - pc) / (pwh / 10)
    t23 = jnp.log(bwh / pwh) * 5
    labs = jnp.sum((jnp.abs(plocs[0:2] - t01)
                    + jnp.abs(plocs[2:4] - t23)).sum(axis=0, keepdims=True)
                   * posf)

    e = jnp.exp(scores)
    ones8 = jnp.zeros((8, num_cls), jnp.float32) + 1.0
    s = jnp.dot(ones8, e, precision=lax.Precision.HIGHEST)[0:1]
    lse = jnp.log(s)
    ci = lax.broadcasted_iota(jnp.int32, (num_cls, num_pri), 0)
    x_t = jnp.sum(jnp.where(ci == label_i, scores, 0.0), axis=0,
                  keepdims=True)
    conf_all = lse - x_t

    cpos = jnp.sum(conf_all * posf)
    conf_neg = jnp.maximum(jnp.where(pos, 0.0, conf_all), 0.0)

    conf_ref[pl.ds(b, 1), :] = conf_neg
    npos_ref[pl.ds(b, 1), :] = n_pos.reshape(1, 1)
    cpos_ref[pl.ds(b, 1), :] = cpos.reshape(1, 1)
    labs_ref[pl.ds(b, 1), :] = labs.reshape(1, 1)

    @pl.when(b == nb - 1)
    def _():
        v = conf_ref[...]
        vi = lax.bitcast_convert_type(v, jnp.int32)
        npos = npos_ref[...]
        k = jnp.minimum(npos * _NEG_POS_RATIO, float(num_pri))

        lo0 = jnp.zeros(npos.shape, jnp.int32)
        hi0 = jnp.full(npos.shape, _F32_INF_BITS, jnp.int32)
        ones_col = jnp.zeros((num_pri, 1), jnp.float32) + 1.0

        def body(_, carry):
            lo, hi = carry
            mid = lo + (hi - lo) // 2
            cnt = jnp.dot((vi >= mid).astype(jnp.float32), ones_col)
            ge = cnt >= k
            return jnp.where(ge, mid, lo), jnp.where(ge, hi, mid)

        lo, _hi = lax.fori_loop(0, 31, body, (lo0, hi0))
        tau = lax.bitcast_convert_type(lo, jnp.float32)
        gt = vi > lo
        cnt_gt = jnp.dot(gt.astype(jnp.float32), ones_col)
        sum_gt = jnp.sum(jnp.where(gt, v, 0.0), axis=1, keepdims=True)
        top_k_sum = sum_gt + (k - cnt_gt) * tau
        top_k_sum = jnp.where(k > 0, top_k_sum, 0.0)

        npos_tot = jnp.sum(npos)
        conf_loss = ((jnp.sum(top_k_sum) + jnp.sum(cpos_ref[...]))
                     / jnp.maximum(npos_tot, 1.0))
        loc_loss = jnp.sum(labs_ref[...]) / jnp.maximum(npos_tot * 4.0, 1.0)
        out_ref[...] = (conf_loss + loc_loss).reshape(1, 1)


@jax.jit
def kernel(predicted_locs, predicted_scores, boxes, labels, priors_cxcy):
    B, P, NC = predicted_scores.shape
    NO = boxes.shape[1]

    scores_t = jnp.transpose(predicted_scores, (0, 2, 1))
    locs_t = jnp.transpose(predicted_locs, (0, 2, 1))
    payload = jnp.concatenate(
        [boxes, labels.astype(jnp.float32)[..., None],
         jnp.zeros((B, NO, 3), jnp.float32)], axis=-1)
    payload_t = jnp.transpose(payload, (0, 2, 1))
    priors_t = priors_cxcy.T

    out = pl.pallas_call(
        _mbl_kernel,
        grid=(B,),
        in_specs=[
            pl.BlockSpec((1, NC, P), lambda b: (b, 0, 0)),
            pl.BlockSpec((1, 4, P), lambda b: (b, 0, 0)),
            pl.BlockSpec((1, 8, NO), lambda b: (b, 0, 0)),
            pl.BlockSpec((1, NO, 4), lambda b: (b, 0, 0)),
            pl.BlockSpec((4, P), lambda b: (0, 0)),
        ],
        out_specs=pl.BlockSpec((1, 1), lambda b: (0, 0)),
        out_shape=jax.ShapeDtypeStruct((1, 1), jnp.float32),
        scratch_shapes=[
            pltpu.VMEM((B, P), jnp.float32),
            pltpu.VMEM((B, 1), jnp.float32),
            pltpu.VMEM((B, 1), jnp.float32),
            pltpu.VMEM((B, 1), jnp.float32),
        ],
        compiler_params=pltpu.CompilerParams(
            dimension_semantics=("arbitrary",)),
    )(scores_t, locs_t, payload_t, boxes, priors_t)
    return out[0, 0]


# speedup vs baseline: 3.9517x; 1.2044x over previous
import jax
import jax.numpy as jnp
from jax import lax
from jax.experimental import pallas as pl
from jax.experimental.pallas import tpu as pltpu

_THRESHOLD = 0.5
_NEG_POS_RATIO = 3.0
_F32_INF_BITS = 0x7F800000


def _match_kernel(locs_ref, payload_ref, boxes_ref, priors_ref,
                  lab_ref, npos_ref, labs_ref):
    plocs = locs_ref[0]
    payload = payload_ref[0]
    boxes = boxes_ref[0]

    pc = priors_ref[0:2, :]
    pwh = priors_ref[2:4, :]
    pcorner0 = pc - pwh / 2
    pcorner1 = pc + pwh / 2
    px0 = pcorner0[0:1, :]
    py0 = pcorner0[1:2, :]
    px1 = pcorner1[0:1, :]
    py1 = pcorner1[1:2, :]
    pa = (px1 - px0) * (py1 - py0)

    bx0 = boxes[:, 0:1]
    by0 = boxes[:, 1:2]
    bx1 = boxes[:, 2:3]
    by1 = boxes[:, 3:4]
    iw = jnp.clip(jnp.minimum(bx1, px1) - jnp.maximum(bx0, px0), 0.0, None)
    ih = jnp.clip(jnp.minimum(by1, py1) - jnp.maximum(by0, py0), 0.0, None)
    inter = iw * ih
    a1 = (bx1 - bx0) * (by1 - by0)
    union = a1 + pa - inter
    iou = inter / union

    vmax = jnp.max(iou, axis=0, keepdims=True)
    onehot = (iou == vmax).astype(jnp.float32)

    g = jnp.dot(payload, onehot, precision=lax.Precision.HIGHEST)
    pos = vmax >= _THRESHOLD
    label_i = jnp.where(pos, (g[4:5] + 0.5).astype(jnp.int32), 0)
    posf = pos.astype(jnp.float32)

    gc0 = g[0:2]
    gc1 = g[2:4]
    bcxy = (gc1 + gc0) / 2
    bwh = gc1 - gc0
    t01 = (bcxy - pc) / (pwh / 10)
    t23 = jnp.log(bwh / pwh) * 5
    labs = jnp.sum((jnp.abs(plocs[0:2] - t01)
                    + jnp.abs(plocs[2:4] - t23)).sum(axis=0, keepdims=True)
                   * posf)

    lab_ref[...] = label_i.reshape(lab_ref.shape)
    npos_ref[...] = jnp.sum(posf).reshape(1, 1, 1)
    labs_ref[...] = labs.reshape(1, 1, 1)


def _conf_kernel(scores_ref, lab_ref, npos_all_ref, labs_all_ref,
                 out_ref, conf_ref, cpos_ref):
    b = pl.program_id(0)
    nb = pl.num_programs(0)

    scores = scores_ref[0]
    num_cls, num_pri = scores.shape
    label_i = lab_ref[0]
    pos = label_i != 0

    e = jnp.exp(scores)
    ones8 = jnp.zeros((8, num_cls), jnp.float32) + 1.0
    s = jnp.dot(ones8, e, precision=lax.Precision.HIGHEST)[0:1]
    lse = jnp.log(s)
    ci = lax.broadcasted_iota(jnp.int32, (num_cls, num_pri), 0)
    x_t = jnp.sum(jnp.where(ci == label_i, scores, 0.0), axis=0,
                  keepdims=True)
    conf_all = lse - x_t

    posf = pos.astype(jnp.float32)
    cpos = jnp.sum(conf_all * posf)
    conf_neg = jnp.maximum(jnp.where(pos, 0.0, conf_all), 0.0)

    conf_ref[pl.ds(b, 1), :] = conf_neg
    cpos_ref[pl.ds(b, 1), :] = cpos.reshape(1, 1)

    @pl.when(b == nb - 1)
    def _():
        v = conf_ref[...]
        vi = lax.bitcast_convert_type(v, jnp.int32)
        npos = npos_all_ref[...][:, :, 0]
        k = jnp.minimum(npos * _NEG_POS_RATIO, float(num_pri))

        lo0 = jnp.zeros(npos.shape, jnp.int32)
        hi0 = jnp.full(npos.shape, _F32_INF_BITS, jnp.int32)

        def body(_, carry):
            lo, hi = carry
            mid = lo + (hi - lo) // 2
            cnt = jnp.sum((vi >= mid).astype(jnp.float32), axis=1,
                          keepdims=True)
            ge = cnt >= k
            return jnp.where(ge, mid, lo), jnp.where(ge, hi, mid)

        lo, _hi = lax.fori_loop(0, 31, body, (lo0, hi0))
        tau = lax.bitcast_convert_type(lo, jnp.float32)
        gt = vi > lo
        cnt_gt = jnp.sum(gt.astype(jnp.float32), axis=1, keepdims=True)
        sum_gt = jnp.sum(jnp.where(gt, v, 0.0), axis=1, keepdims=True)
        top_k_sum = sum_gt + (k - cnt_gt) * tau
        top_k_sum = jnp.where(k > 0, top_k_sum, 0.0)

        npos_tot = jnp.sum(npos)
        conf_loss = ((jnp.sum(top_k_sum) + jnp.sum(cpos_ref[...]))
                     / jnp.maximum(npos_tot, 1.0))
        loc_loss = (jnp.sum(labs_all_ref[...])
                    / jnp.maximum(npos_tot * 4.0, 1.0))
        out_ref[...] = (conf_loss + loc_loss).reshape(1, 1)


@jax.jit
def kernel(predicted_locs, predicted_scores, boxes, labels, priors_cxcy):
    B, P, NC = predicted_scores.shape
    NO = boxes.shape[1]

    scores_t = jnp.transpose(predicted_scores, (0, 2, 1))
    locs_t = jnp.transpose(predicted_locs, (0, 2, 1))
    payload = jnp.concatenate(
        [boxes, labels.astype(jnp.float32)[..., None],
         jnp.zeros((B, NO, 3), jnp.float32)], axis=-1)
    payload_t = jnp.transpose(payload, (0, 2, 1))
    priors_t = priors_cxcy.T

    lab, nposs, labss = pl.pallas_call(
        _match_kernel,
        grid=(B,),
        in_specs=[
            pl.BlockSpec((1, 4, P), lambda b: (b, 0, 0)),
            pl.BlockSpec((1, 8, NO), lambda b: (b, 0, 0)),
            pl.BlockSpec((1, NO, 4), lambda b: (b, 0, 0)),
            pl.BlockSpec((4, P), lambda b: (0, 0)),
        ],
        out_specs=[
            pl.BlockSpec((1, 1, P), lambda b: (b, 0, 0)),
            pl.BlockSpec((1, 1, 1), lambda b: (b, 0, 0)),
            pl.BlockSpec((1, 1, 1), lambda b: (b, 0, 0)),
        ],
        out_shape=[
            jax.ShapeDtypeStruct((B, 1, P), jnp.int32),
            jax.ShapeDtypeStruct((B, 1, 1), jnp.float32),
            jax.ShapeDtypeStruct((B, 1, 1), jnp.float32),
        ],
        compiler_params=pltpu.CompilerParams(
            dimension_semantics=("arbitrary",)),
    )(locs_t, payload_t, boxes, priors_t)

    out = pl.pallas_call(
        _conf_kernel,
        grid=(B,),
        in_specs=[
            pl.BlockSpec((1, NC, P), lambda b: (b, 0, 0)),
            pl.BlockSpec((1, 1, P), lambda b: (b, 0, 0)),
            pl.BlockSpec((B, 1, 1), lambda b: (0, 0, 0)),
            pl.BlockSpec((B, 1, 1), lambda b: (0, 0, 0)),
        ],
        out_specs=pl.BlockSpec((1, 1), lambda b: (0, 0)),
        out_shape=jax.ShapeDtypeStruct((1, 1), jnp.float32),
        scratch_shapes=[
            pltpu.VMEM((B, P), jnp.float32),
            pltpu.VMEM((B, 1), jnp.float32),
        ],
        compiler_params=pltpu.CompilerParams(
            dimension_semantics=("arbitrary",)),
    )(scores_t, lab, nposs, labss)
    return out[0, 0]
